# baseline (speedup 1.0000x reference)
import functools

import jax
import jax.numpy as jnp
from jax import lax
from jax.experimental import pallas as pl
from jax.experimental.pallas import tpu as pltpu
from jax.experimental.pallas import tpu_sc as plsc

N_NODES = 10000
N_EDGES = 320000
D = 128

NC = 2
NS = 16
NW = NC * NS
EPW = N_EDGES // NW
CHUNK = 40
NCHUNK = EPW // CHUNK
KBUF = 6
SBCH = 48
NSB = NCHUNK // SBCH
TAIL = NCHUNK - NSB * SBCH
RCHUNK = 40
NRCHUNK = N_NODES // RCHUNK
RITER = -(-NRCHUNK // NS)


def _linear_body(x_ref, w_ref, b_ref, o_ref):
    o_ref[...] = lax.dot_general(
        x_ref[...], w_ref[...],
        dimension_numbers=(((1,), (1,)), ((), ())),
        preferred_element_type=jnp.float32,
    ) + b_ref[...]


def _linear(x, W, b):
    grid = 10
    bm = N_NODES // grid
    return pl.pallas_call(
        _linear_body,
        grid=(grid,),
        in_specs=[
            pl.BlockSpec((bm, D), lambda i: (i, 0)),
            pl.BlockSpec((D, D), lambda i: (0, 0)),
            pl.BlockSpec((1, D), lambda i: (0, 0)),
        ],
        out_specs=pl.BlockSpec((bm, D), lambda i: (i, 0)),
        out_shape=jax.ShapeDtypeStruct((N_NODES, D), jnp.float32),
    )(x, W, b.reshape(1, D))


def _spmm_body(support, src, dst, part,
               srcb, dstb, r0, r1, r2, r3, r4, r5, acc, gsem, ssem, isem):
    c = lax.axis_index("c")
    s = lax.axis_index("s")
    w = c * NS + s
    rows = [r0, r1, r2, r3, r4, r5]

    def _zf(i, carry):
        r0[i // 8, pl.ds((i % 8) * 16, 16)] = jnp.zeros((16,), jnp.float32)
        return carry
    lax.fori_loop(0, RCHUNK * (D // 16), _zf, 0)
    zsrc = r0.at[pl.ds(0, RCHUNK)]
    for t in range(RITER):
        k = s + t * NS
        @pl.when(k < NRCHUNK)
        def _():
            pltpu.sync_copy(zsrc, acc.at[pl.ds(k * RCHUNK, RCHUNK)])

    plsc.subcore_barrier()

    def _drain_scatter(b):
        pltpu.make_async_copy(rows[b], acc.at[pl.ds(0, CHUNK)],
                              ssem.at[b]).wait()

    def _edge_iter(it, carry):
        j0 = it * KBUF
        gd = []
        for b in range(KBUF):
            @pl.when(it > 0)
            def _():
                _drain_scatter(b)
            gd.append(pltpu.async_copy(support.at[srcb.at[j0 + b]], rows[b],
                                       gsem.at[b]))
        for b in range(KBUF):
            gd[b].wait()
            pltpu.async_copy(rows[b], acc.at[dstb.at[j0 + b]],
                             ssem.at[b], add=True)
        return carry

    pltpu.sync_copy(src.at[w, pl.ds(0, SBCH)], srcb)
    pltpu.sync_copy(dst.at[w, pl.ds(0, SBCH)], dstb)
    for blk in range(NSB):
        lax.fori_loop(0, SBCH // KBUF, _edge_iter, 0)
        nxt = (blk + 1) * SBCH
        nn = SBCH if blk + 1 < NSB else TAIL
        isd = [pltpu.async_copy(src.at[w, pl.ds(nxt, nn)],
                                srcb.at[pl.ds(0, nn)], isem),
               pltpu.async_copy(dst.at[w, pl.ds(nxt, nn)],
                                dstb.at[pl.ds(0, nn)], isem)]
        for b in range(KBUF):
            _drain_scatter(b)
        for d in isd:
            d.wait()
    for j in range(TAIL):
        jj = jnp.int32(j)
        pltpu.async_copy(support.at[srcb.at[jj]], r0, gsem.at[0]).wait()
        pltpu.sync_copy(r0, acc.at[dstb.at[jj]], add=True)
    plsc.subcore_barrier()

    for t in range(RITER):
        k = s + t * NS
        @pl.when(k < NRCHUNK)
        def _():
            pltpu.sync_copy(acc.at[pl.ds(k * RCHUNK, RCHUNK)],
                            part.at[c, pl.ds(k * RCHUNK, RCHUNK)])


def _sc_spmm(support, src, dst):
    mesh = plsc.VectorSubcoreMesh(core_axis_name="c", subcore_axis_name="s")
    f = pl.kernel(
        _spmm_body,
        out_type=jax.ShapeDtypeStruct((NC, N_NODES, D), jnp.float32),
        name="sc_spmm",
        mesh=mesh,
        scratch_types=[
            pltpu.VMEM((SBCH, CHUNK), jnp.int32),
            pltpu.VMEM((SBCH, CHUNK), jnp.int32),
            pltpu.VMEM((CHUNK, D), jnp.float32),
            pltpu.VMEM((CHUNK, D), jnp.float32),
            pltpu.VMEM((CHUNK, D), jnp.float32),
            pltpu.VMEM((CHUNK, D), jnp.float32),
            pltpu.VMEM((CHUNK, D), jnp.float32),
            pltpu.VMEM((CHUNK, D), jnp.float32),
        ] + [
            pltpu.VMEM_SHARED((N_NODES, D), jnp.float32),
            pltpu.SemaphoreType.DMA((KBUF,)),
            pltpu.SemaphoreType.DMA((KBUF,)),
            pltpu.SemaphoreType.DMA,
        ],
    )
    return f(support, src, dst)


def _combine_body(p_ref, o_ref):
    o_ref[...] = p_ref[0] + p_ref[1]


def _combine(partials):
    grid = 10
    bm = N_NODES // grid
    return pl.pallas_call(
        _combine_body,
        grid=(grid,),
        in_specs=[pl.BlockSpec((NC, bm, D), lambda i: (0, i, 0))],
        out_specs=pl.BlockSpec((bm, D), lambda i: (i, 0)),
        out_shape=jax.ShapeDtypeStruct((N_NODES, D), jnp.float32),
    )(partials)


def kernel(graph, x, W, b):
    src = graph[0].astype(jnp.int32).reshape(NW, NCHUNK, CHUNK)
    dst = graph[1].astype(jnp.int32).reshape(NW, NCHUNK, CHUNK)
    support = _linear(x, W, b)
    partials = _sc_spmm(support, src, dst)
    return _combine(partials)


# speedup vs baseline: 1.0746x; 1.0746x over previous
import functools

import jax
import jax.numpy as jnp
from jax import lax
from jax.experimental import pallas as pl
from jax.experimental.pallas import tpu as pltpu
from jax.experimental.pallas import tpu_sc as plsc

N_NODES = 10000
N_EDGES = 320000
D = 128

NC = 2
NS = 16
NW = NC * NS
EPW = N_EDGES // NW
CHUNK = 40
NCHUNK = EPW // CHUNK
KBUF = 6
SBCH = 48
NSB = NCHUNK // SBCH
TAIL = NCHUNK - NSB * SBCH
RCHUNK = 40
NRCHUNK = N_NODES // RCHUNK
RITER = -(-NRCHUNK // NS)


def _linear_body(x_ref, w_ref, b_ref, o_ref):
    o_ref[...] = lax.dot_general(
        x_ref[...], w_ref[...],
        dimension_numbers=(((1,), (1,)), ((), ())),
        preferred_element_type=jnp.float32,
    ) + b_ref[...]


def _linear(x, W, b):
    grid = 10
    bm = N_NODES // grid
    return pl.pallas_call(
        _linear_body,
        grid=(grid,),
        in_specs=[
            pl.BlockSpec((bm, D), lambda i: (i, 0)),
            pl.BlockSpec((D, D), lambda i: (0, 0)),
            pl.BlockSpec((1, D), lambda i: (0, 0)),
        ],
        out_specs=pl.BlockSpec((bm, D), lambda i: (i, 0)),
        out_shape=jax.ShapeDtypeStruct((N_NODES, D), jnp.float32),
    )(x, W, b.reshape(1, D))


def _spmm_body(support, src, dst, part,
               srcb, dstb, r0, r1, r2, r3, r4, r5, acc,
               gsem, ssem, isem, zsem):
    c = lax.axis_index("c")
    s = lax.axis_index("s")
    w = c * NS + s
    rows = [r0, r1, r2, r3, r4, r5]

    isd0 = [pltpu.async_copy(src.at[w, pl.ds(0, SBCH)], srcb, isem),
            pltpu.async_copy(dst.at[w, pl.ds(0, SBCH)], dstb, isem)]

    def _zf(i, carry):
        r0[i // 8, pl.ds((i % 8) * 16, 16)] = jnp.zeros((16,), jnp.float32)
        return carry
    lax.fori_loop(0, RCHUNK * (D // 16), _zf, 0)
    zsrc = r0.at[pl.ds(0, RCHUNK)]
    for t in range(RITER):
        k = s + t * NS
        @pl.when(k < NRCHUNK)
        def _():
            pltpu.async_copy(zsrc, acc.at[pl.ds(k * RCHUNK, RCHUNK)], zsem)
    for t in range(RITER):
        k = s + t * NS
        @pl.when(k < NRCHUNK)
        def _():
            pltpu.make_async_copy(zsrc, acc.at[pl.ds(0, RCHUNK)], zsem).wait()
    for d in isd0:
        d.wait()

    plsc.subcore_barrier()

    def _drain_scatter(b):
        pltpu.make_async_copy(rows[b], acc.at[pl.ds(0, CHUNK)],
                              ssem.at[b]).wait()

    def _edge_iter(it, carry):
        j0 = it * KBUF
        gd = []
        for b in range(KBUF):
            @pl.when(it > 0)
            def _():
                _drain_scatter(b)
            gd.append(pltpu.async_copy(support.at[srcb.at[j0 + b]], rows[b],
                                       gsem.at[b]))
        for b in range(KBUF):
            gd[b].wait()
            pltpu.async_copy(rows[b], acc.at[dstb.at[j0 + b]],
                             ssem.at[b], add=True)
        return carry

    for blk in range(NSB):
        lax.fori_loop(0, SBCH // KBUF, _edge_iter, 0)
        nxt = (blk + 1) * SBCH
        nn = SBCH if blk + 1 < NSB else TAIL
        isd = [pltpu.async_copy(src.at[w, pl.ds(nxt, nn)],
                                srcb.at[pl.ds(0, nn)], isem),
               pltpu.async_copy(dst.at[w, pl.ds(nxt, nn)],
                                dstb.at[pl.ds(0, nn)], isem)]
        for b in range(KBUF):
            _drain_scatter(b)
        for d in isd:
            d.wait()
    for p0 in range(0, TAIL, KBUF):
        phase = range(p0, min(p0 + KBUF, TAIL))
        tgd = [pltpu.async_copy(support.at[srcb.at[jnp.int32(j)]],
                                rows[j % KBUF], gsem.at[j % KBUF])
               for j in phase]
        for i, j in enumerate(phase):
            tgd[i].wait()
            pltpu.async_copy(rows[j % KBUF], acc.at[dstb.at[jnp.int32(j)]],
                             ssem.at[j % KBUF], add=True)
        for j in phase:
            _drain_scatter(j % KBUF)
    plsc.subcore_barrier()

    for t in range(RITER):
        k = s + t * NS
        @pl.when(k < NRCHUNK)
        def _():
            pltpu.async_copy(acc.at[pl.ds(k * RCHUNK, RCHUNK)],
                             part.at[c, pl.ds(k * RCHUNK, RCHUNK)], zsem)
    for t in range(RITER):
        k = s + t * NS
        @pl.when(k < NRCHUNK)
        def _():
            pltpu.make_async_copy(acc.at[pl.ds(0, RCHUNK)],
                                  part.at[c, pl.ds(0, RCHUNK)], zsem).wait()


def _sc_spmm(support, src, dst):
    mesh = plsc.VectorSubcoreMesh(core_axis_name="c", subcore_axis_name="s")
    f = pl.kernel(
        _spmm_body,
        out_type=jax.ShapeDtypeStruct((NC, N_NODES, D), jnp.float32),
        name="sc_spmm",
        mesh=mesh,
        scratch_types=[
            pltpu.VMEM((SBCH, CHUNK), jnp.int32),
            pltpu.VMEM((SBCH, CHUNK), jnp.int32),
            pltpu.VMEM((CHUNK, D), jnp.float32),
            pltpu.VMEM((CHUNK, D), jnp.float32),
            pltpu.VMEM((CHUNK, D), jnp.float32),
            pltpu.VMEM((CHUNK, D), jnp.float32),
            pltpu.VMEM((CHUNK, D), jnp.float32),
            pltpu.VMEM((CHUNK, D), jnp.float32),
        ] + [
            pltpu.VMEM_SHARED((N_NODES, D), jnp.float32),
            pltpu.SemaphoreType.DMA((KBUF,)),
            pltpu.SemaphoreType.DMA((KBUF,)),
            pltpu.SemaphoreType.DMA,
            pltpu.SemaphoreType.DMA,
        ],
    )
    return f(support, src, dst)


def _combine_body(p_ref, o_ref):
    o_ref[...] = p_ref[0] + p_ref[1]


def _combine(partials):
    grid = 10
    bm = N_NODES // grid
    return pl.pallas_call(
        _combine_body,
        grid=(grid,),
        in_specs=[pl.BlockSpec((NC, bm, D), lambda i: (0, i, 0))],
        out_specs=pl.BlockSpec((bm, D), lambda i: (i, 0)),
        out_shape=jax.ShapeDtypeStruct((N_NODES, D), jnp.float32),
    )(partials)


def kernel(graph, x, W, b):
    src = graph[0].astype(jnp.int32).reshape(NW, NCHUNK, CHUNK)
    dst = graph[1].astype(jnp.int32).reshape(NW, NCHUNK, CHUNK)
    support = _linear(x, W, b)
    partials = _sc_spmm(support, src, dst)
    return _combine(partials)
